# baseline (speedup 1.0000x reference)
import functools

import jax
import jax.numpy as jnp
from jax import lax
from jax.experimental import pallas as pl
from jax.experimental.pallas import tpu as pltpu
from jax.experimental.pallas import tpu_sc as plsc

BATCH = 16384
DIM = 32
NC = 2
NS = 16
L = 16
NW = NC * NS
BPW = BATCH // NW
GROUPS = BPW // L


def _bpr_body(user_hbm, ii_hbm, ij_hbm, U_hbm, I_hbm, oi_hbm, oj_hbm,
              uidx_v, iidx_v, jidx_v, urows_v, virows_v, vjrows_v,
              oi_v, oj_v, sem):
    wid = lax.axis_index("s") * NC + lax.axis_index("c")
    base = wid * BPW

    pltpu.sync_copy(user_hbm.at[pl.ds(base, BPW)], uidx_v)
    pltpu.sync_copy(ii_hbm.at[pl.ds(base, BPW)], iidx_v)
    pltpu.sync_copy(ij_hbm.at[pl.ds(base, BPW)], jidx_v)

    cu = pltpu.async_copy(U_hbm.at[uidx_v], urows_v, sem)
    ci = pltpu.async_copy(I_hbm.at[iidx_v], virows_v, sem)
    cj = pltpu.async_copy(I_hbm.at[jidx_v], vjrows_v, sem)
    cu.wait()
    ci.wait()
    cj.wait()

    def group(g, _):
        rows = lax.iota(jnp.int32, L) + g * L
        acc_i = jnp.zeros((L,), jnp.float32)
        acc_j = jnp.zeros((L,), jnp.float32)
        for d in range(DIM):
            col = jnp.full((L,), d, jnp.int32)
            uc = plsc.load_gather(urows_v, [rows, col])
            acc_i = acc_i + uc * plsc.load_gather(virows_v, [rows, col])
            acc_j = acc_j + uc * plsc.load_gather(vjrows_v, [rows, col])
        oi_v[pl.ds(g * L, L)] = acc_i
        oj_v[pl.ds(g * L, L)] = acc_j
        return 0

    lax.fori_loop(0, GROUPS, group, 0)

    pltpu.sync_copy(oi_v, oi_hbm.at[pl.ds(base, BPW)])
    pltpu.sync_copy(oj_v, oj_hbm.at[pl.ds(base, BPW)])


@jax.jit
def kernel(user, item_i, item_j, U, I):
    mesh = plsc.VectorSubcoreMesh(core_axis_name="c", subcore_axis_name="s")
    f = pl.kernel(
        _bpr_body,
        out_type=(
            jax.ShapeDtypeStruct((BATCH,), jnp.float32),
            jax.ShapeDtypeStruct((BATCH,), jnp.float32),
        ),
        mesh=mesh,
        scratch_types=[
            pltpu.VMEM((BPW,), jnp.int32),
            pltpu.VMEM((BPW,), jnp.int32),
            pltpu.VMEM((BPW,), jnp.int32),
            pltpu.VMEM((BPW, DIM), jnp.float32),
            pltpu.VMEM((BPW, DIM), jnp.float32),
            pltpu.VMEM((BPW, DIM), jnp.float32),
            pltpu.VMEM((BPW,), jnp.float32),
            pltpu.VMEM((BPW,), jnp.float32),
            pltpu.SemaphoreType.DMA,
        ],
        compiler_params=pltpu.CompilerParams(
            needs_layout_passes=False, use_tc_tiling_on_sc=False),
    )
    pred_i, pred_j = f(user.astype(jnp.int32), item_i.astype(jnp.int32),
                       item_j.astype(jnp.int32), U, I)
    return (pred_i, pred_j)


# speedup vs baseline: 1.4456x; 1.4456x over previous
import jax
import jax.numpy as jnp
from jax import lax
from jax.experimental import pallas as pl
from jax.experimental.pallas import tpu as pltpu
from jax.experimental.pallas import tpu_sc as plsc

BATCH = 16384
DIM = 32
NC = 2
NS = 16
L = 16
NW = NC * NS
BPW = BATCH // NW
RC = 32
NCHUNK = BPW // RC


def _bpr_body(user_hbm, ii_hbm, ij_hbm, U_hbm, I_hbm, oi_hbm, oj_hbm,
              uidx_v, iidx_v, jidx_v, urows_v, virows_v, vjrows_v,
              oi_v, oj_v, sem):
    wid = lax.axis_index("s") * NC + lax.axis_index("c")
    base = wid * BPW

    pltpu.sync_copy(user_hbm.at[pl.ds(base, BPW)], uidx_v)
    pltpu.sync_copy(ii_hbm.at[pl.ds(base, BPW)], iidx_v)
    pltpu.sync_copy(ij_hbm.at[pl.ds(base, BPW)], jidx_v)

    def chunk(c, _):
        row0 = c * RC
        for r in range(RC):
            if r % L == 0:
                uvec = uidx_v[pl.ds(row0 + r, L)]
                ivec = iidx_v[pl.ds(row0 + r, L)]
                jvec = jidx_v[pl.ds(row0 + r, L)]
            pltpu.async_copy(U_hbm.at[pl.ds(uvec[r % L], 1), :],
                             urows_v.at[pl.ds(r, 1), :], sem)
            pltpu.async_copy(I_hbm.at[pl.ds(ivec[r % L], 1), :],
                             virows_v.at[pl.ds(r, 1), :], sem)
            pltpu.async_copy(I_hbm.at[pl.ds(jvec[r % L], 1), :],
                             vjrows_v.at[pl.ds(r, 1), :], sem)
        for r in range(RC):
            pltpu.make_async_copy(U_hbm.at[pl.ds(0, 1), :],
                                  urows_v.at[pl.ds(r, 1), :], sem).wait()
            pltpu.make_async_copy(I_hbm.at[pl.ds(0, 1), :],
                                  virows_v.at[pl.ds(r, 1), :], sem).wait()
            pltpu.make_async_copy(I_hbm.at[pl.ds(0, 1), :],
                                  vjrows_v.at[pl.ds(r, 1), :], sem).wait()

        for g in range(RC // L):
            rows = lax.iota(jnp.int32, L) + g * L
            acc_i = jnp.zeros((L,), jnp.float32)
            acc_j = jnp.zeros((L,), jnp.float32)
            for d in range(DIM):
                col = jnp.full((L,), d, jnp.int32)
                uc = plsc.load_gather(urows_v, [rows, col])
                acc_i = acc_i + uc * plsc.load_gather(virows_v, [rows, col])
                acc_j = acc_j + uc * plsc.load_gather(vjrows_v, [rows, col])
            oi_v[pl.ds(row0 + g * L, L)] = acc_i
            oj_v[pl.ds(row0 + g * L, L)] = acc_j
        return 0

    lax.fori_loop(0, NCHUNK, chunk, 0)

    pltpu.sync_copy(oi_v, oi_hbm.at[pl.ds(base, BPW)])
    pltpu.sync_copy(oj_v, oj_hbm.at[pl.ds(base, BPW)])


@jax.jit
def kernel(user, item_i, item_j, U, I):
    mesh = plsc.VectorSubcoreMesh(core_axis_name="c", subcore_axis_name="s")
    f = pl.kernel(
        _bpr_body,
        out_type=(
            jax.ShapeDtypeStruct((BATCH,), jnp.float32),
            jax.ShapeDtypeStruct((BATCH,), jnp.float32),
        ),
        mesh=mesh,
        scratch_types=[
            pltpu.VMEM((BPW,), jnp.int32),
            pltpu.VMEM((BPW,), jnp.int32),
            pltpu.VMEM((BPW,), jnp.int32),
            pltpu.VMEM((RC, DIM), jnp.float32),
            pltpu.VMEM((RC, DIM), jnp.float32),
            pltpu.VMEM((RC, DIM), jnp.float32),
            pltpu.VMEM((BPW,), jnp.float32),
            pltpu.VMEM((BPW,), jnp.float32),
            pltpu.SemaphoreType.DMA,
        ],
        compiler_params=pltpu.CompilerParams(needs_layout_passes=False),
    )
    pred_i, pred_j = f(user.astype(jnp.int32), item_i.astype(jnp.int32),
                       item_j.astype(jnp.int32), U, I)
    return (pred_i, pred_j)


# speedup vs baseline: 1.4737x; 1.0195x over previous
import jax
import jax.numpy as jnp
from jax import lax
from jax.experimental import pallas as pl
from jax.experimental.pallas import tpu as pltpu
from jax.experimental.pallas import tpu_sc as plsc

BATCH = 16384
DIM = 32
NC = 2
NS = 16
L = 16
NW = NC * NS
BPW = BATCH // NW
RC = 16
NCHUNK = BPW // RC


def _bpr_body(user_hbm, ii_hbm, ij_hbm, U_hbm, I_hbm, oi_hbm, oj_hbm,
              uidx_v, iidx_v, jidx_v,
              ua_v, via_v, vja_v, ub_v, vib_v, vjb_v,
              oi_v, oj_v, sem_a, sem_b):
    wid = lax.axis_index("s") * NC + lax.axis_index("c")
    base = wid * BPW

    pltpu.sync_copy(user_hbm.at[pl.ds(base, BPW)], uidx_v)
    pltpu.sync_copy(ii_hbm.at[pl.ds(base, BPW)], iidx_v)
    pltpu.sync_copy(ij_hbm.at[pl.ds(base, BPW)], jidx_v)

    def fire(c, ur, vir, vjr, sem):
        row0 = c * RC
        uvec = uidx_v[pl.ds(row0, L)]
        ivec = iidx_v[pl.ds(row0, L)]
        jvec = jidx_v[pl.ds(row0, L)]
        for r in range(RC):
            pltpu.async_copy(U_hbm.at[pl.ds(uvec[r], 1), :],
                             ur.at[pl.ds(r, 1), :], sem)
            pltpu.async_copy(I_hbm.at[pl.ds(ivec[r], 1), :],
                             vir.at[pl.ds(r, 1), :], sem)
            pltpu.async_copy(I_hbm.at[pl.ds(jvec[r], 1), :],
                             vjr.at[pl.ds(r, 1), :], sem)

    def drain_compute(c, ur, vir, vjr, sem):
        row0 = c * RC
        for r in range(RC):
            pltpu.make_async_copy(U_hbm.at[pl.ds(0, 1), :],
                                  ur.at[pl.ds(r, 1), :], sem).wait()
            pltpu.make_async_copy(I_hbm.at[pl.ds(0, 1), :],
                                  vir.at[pl.ds(r, 1), :], sem).wait()
            pltpu.make_async_copy(I_hbm.at[pl.ds(0, 1), :],
                                  vjr.at[pl.ds(r, 1), :], sem).wait()
        rows = lax.iota(jnp.int32, L)
        acc_i = jnp.zeros((L,), jnp.float32)
        acc_j = jnp.zeros((L,), jnp.float32)
        for d in range(DIM):
            col = jnp.full((L,), d, jnp.int32)
            uc = plsc.load_gather(ur, [rows, col])
            acc_i = acc_i + uc * plsc.load_gather(vir, [rows, col])
            acc_j = acc_j + uc * plsc.load_gather(vjr, [rows, col])
        oi_v[pl.ds(row0, L)] = acc_i
        oj_v[pl.ds(row0, L)] = acc_j

    def step(g, _):
        fire(2 * g, ua_v, via_v, vja_v, sem_a)

        @pl.when(g > 0)
        def _():
            drain_compute(2 * g - 1, ub_v, vib_v, vjb_v, sem_b)

        fire(2 * g + 1, ub_v, vib_v, vjb_v, sem_b)
        drain_compute(2 * g, ua_v, via_v, vja_v, sem_a)
        return 0

    lax.fori_loop(0, NCHUNK // 2, step, 0)
    drain_compute(NCHUNK - 1, ub_v, vib_v, vjb_v, sem_b)

    pltpu.sync_copy(oi_v, oi_hbm.at[pl.ds(base, BPW)])
    pltpu.sync_copy(oj_v, oj_hbm.at[pl.ds(base, BPW)])


@jax.jit
def kernel(user, item_i, item_j, U, I):
    mesh = plsc.VectorSubcoreMesh(core_axis_name="c", subcore_axis_name="s")
    f = pl.kernel(
        _bpr_body,
        out_type=(
            jax.ShapeDtypeStruct((BATCH,), jnp.float32),
            jax.ShapeDtypeStruct((BATCH,), jnp.float32),
        ),
        mesh=mesh,
        scratch_types=[
            pltpu.VMEM((BPW,), jnp.int32),
            pltpu.VMEM((BPW,), jnp.int32),
            pltpu.VMEM((BPW,), jnp.int32),
            pltpu.VMEM((RC, DIM), jnp.float32),
            pltpu.VMEM((RC, DIM), jnp.float32),
            pltpu.VMEM((RC, DIM), jnp.float32),
            pltpu.VMEM((RC, DIM), jnp.float32),
            pltpu.VMEM((RC, DIM), jnp.float32),
            pltpu.VMEM((RC, DIM), jnp.float32),
            pltpu.VMEM((BPW,), jnp.float32),
            pltpu.VMEM((BPW,), jnp.float32),
            pltpu.SemaphoreType.DMA,
            pltpu.SemaphoreType.DMA,
        ],
        compiler_params=pltpu.CompilerParams(needs_layout_passes=False),
    )
    pred_i, pred_j = f(user.astype(jnp.int32), item_i.astype(jnp.int32),
                       item_j.astype(jnp.int32), U, I)
    return (pred_i, pred_j)
